# baseline (speedup 1.0000x reference)
import functools

import jax
import jax.numpy as jnp
from jax import lax
from jax.experimental import pallas as pl
from jax.experimental.pallas import tpu as pltpu
from jax.experimental.pallas import tpu_sc as plsc

D = 64
NC = 2
NS = 16
NW = NC * NS
L = 16
B = 128
NBH = 4


@functools.partial(jax.jit, static_argnames=("n", "s"))
def _embed_gather(x5, table, *, n, s):
    nbh_tot = n // B
    sp8 = (s + 7) // 8
    mesh = plsc.VectorSubcoreMesh(core_axis_name="c", subcore_axis_name="s")

    def body(x_hbm, table_hbm, out_hbm, idx_v, *rest):
        rows = rest[:NBH]
        tbuf = rest[NBH:2 * NBH]
        gsem = rest[2 * NBH:3 * NBH]
        osem = rest[3 * NBH:4 * NBH]
        wid = lax.axis_index("s") * NC + lax.axis_index("c")
        nh0 = wid * NBH

        pltpu.sync_copy(x_hbm.at[:, pl.ds(nh0, NBH)], idx_v)

        iota = lax.iota(jnp.int32, L)
        rowv = [iota + (g * L) for g in range(B // L)]

        def g_start(s_, b):
            pltpu.async_copy(
                table_hbm.at[idx_v.at[s_ // 8, b, s_ % 8]], rows[b], gsem[b])

        def g_wait(s_, b):
            pltpu.make_async_copy(
                table_hbm.at[idx_v.at[s_ // 8, b, s_ % 8]], rows[b],
                gsem[b]).wait()

        def o_start(s_, b):
            pltpu.async_copy(
                tbuf[b], out_hbm.at[s_, :, nh0 + b], osem[b])

        def o_wait(s_, b):
            pltpu.make_async_copy(
                tbuf[b], out_hbm.at[s_, :, nh0 + b], osem[b]).wait()

        def transpose(b):
            for g in range(B // L):
                rv = rowv[g]
                for c in range(D):
                    cv = jnp.full((L,), c, jnp.int32)
                    tbuf[b][c // 8, c % 8, pl.ds(g * L, L)] = (
                        plsc.load_gather(rows[b], [rv, cv]))

        for b in range(NBH):
            g_start(0, b)

        @pl.loop(0, s)
        def _outer(s_):
            for b in range(NBH):
                g_wait(s_, b)

                @pl.when(s_ > 0)
                def _():
                    o_wait(s_ - 1, b)

                transpose(b)

                @pl.when(s_ < s - 1)
                def _():
                    g_start(s_ + 1, b)

                o_start(s_, b)

        for b in range(NBH):
            o_wait(s - 1, b)

    call = pl.kernel(
        body,
        out_type=jax.ShapeDtypeStruct((s, 8, nbh_tot, 8, B), jnp.float32),
        mesh=mesh,
        scratch_types=(
            [pltpu.VMEM((sp8, NBH, 8, B), jnp.int32)]
            + [pltpu.VMEM((B, D), jnp.float32) for _ in range(NBH)]
            + [pltpu.VMEM((8, 8, B), jnp.float32) for _ in range(NBH)]
            + [pltpu.SemaphoreType.DMA for _ in range(2 * NBH)]
        ),
        compiler_params=pltpu.CompilerParams(
            use_tc_tiling_on_sc=False, needs_layout_passes=False),
    )
    return call(x5, table)


def kernel(x, phychem):
    n, s = x.shape
    sp = (s + 7) // 8 * 8
    xp = jnp.pad(x, ((0, 0), (0, sp - s)))
    x5 = xp.reshape(n // B, B, sp // 8, 8).transpose(2, 0, 3, 1)
    out5 = _embed_gather(x5, phychem, n=n, s=s)
    return out5.transpose(2, 4, 0, 1, 3).reshape(n, s, D)


# speedup vs baseline: 2.2239x; 2.2239x over previous
import functools

import jax
import jax.numpy as jnp
from jax import lax
from jax.experimental import pallas as pl
from jax.experimental.pallas import tpu as pltpu
from jax.experimental.pallas import tpu_sc as plsc

D = 64
NC = 2
NS = 16
NW = NC * NS
L = 16
B = 128
NBH = 4


@functools.partial(jax.jit, static_argnames=("n", "s"))
def _embed_gather(x5, table, *, n, s):
    nbh_tot = n // B
    sp8 = (s + 7) // 8
    mesh = plsc.VectorSubcoreMesh(core_axis_name="c", subcore_axis_name="s")

    def body(x_hbm, table_hbm, out_hbm, idx_v, *rest):
        rows = rest[:NBH]
        tbuf = rest[NBH:2 * NBH]
        gsem = rest[2 * NBH:3 * NBH]
        osem = rest[3 * NBH:4 * NBH]
        wid = lax.axis_index("s") * NC + lax.axis_index("c")
        nh0 = wid * NBH

        pltpu.sync_copy(x_hbm.at[:, pl.ds(nh0, NBH)], idx_v)

        iota = lax.iota(jnp.int32, L)
        cperm = [jnp.bitwise_and(iota + k, L - 1) for k in range(L)]

        def g_start(s_, b):
            pltpu.async_copy(
                table_hbm.at[idx_v.at[s_ // 8, b, s_ % 8]], rows[b], gsem[b])

        def g_wait(s_, b):
            pltpu.make_async_copy(
                table_hbm.at[idx_v.at[s_ // 8, b, s_ % 8]], rows[b],
                gsem[b]).wait()

        def o_start(s_, b):
            pltpu.async_copy(
                tbuf[b], out_hbm.at[s_, :, nh0 + b], osem[b])

        def o_wait(s_, b):
            pltpu.make_async_copy(
                tbuf[b], out_hbm.at[s_, :, nh0 + b], osem[b]).wait()

        def transpose(b):
            @pl.loop(0, B // L)
            def _g(g):
                rv = iota + g * L
                for cb in range(D // L):
                    for k in range(L):
                        cp = cperm[k]
                        cv = cp + (cb * L)
                        v = plsc.load_gather(rows[b], [rv, cv])
                        plsc.store_scatter(
                            tbuf[b],
                            [jnp.right_shift(cv, 3), jnp.bitwise_and(cp, 7),
                             rv], v)

        for b in range(NBH):
            g_start(0, b)

        @pl.loop(0, s)
        def _outer(s_):
            for b in range(NBH):
                g_wait(s_, b)

                @pl.when(s_ > 0)
                def _():
                    o_wait(s_ - 1, b)

                transpose(b)

                @pl.when(s_ < s - 1)
                def _():
                    g_start(s_ + 1, b)

                o_start(s_, b)

        for b in range(NBH):
            o_wait(s - 1, b)

    call = pl.kernel(
        body,
        out_type=jax.ShapeDtypeStruct((s, 8, nbh_tot, 8, B), jnp.float32),
        mesh=mesh,
        scratch_types=(
            [pltpu.VMEM((sp8, NBH, 8, B), jnp.int32)]
            + [pltpu.VMEM((B, D), jnp.float32) for _ in range(NBH)]
            + [pltpu.VMEM((8, 8, B), jnp.float32) for _ in range(NBH)]
            + [pltpu.SemaphoreType.DMA for _ in range(2 * NBH)]
        ),
        compiler_params=pltpu.CompilerParams(
            use_tc_tiling_on_sc=False, needs_layout_passes=False),
    )
    return call(x5, table)


def kernel(x, phychem):
    n, s = x.shape
    sp = (s + 7) // 8 * 8
    xp = jnp.pad(x, ((0, 0), (0, sp - s)))
    x5 = xp.reshape(n // B, B, sp // 8, 8).transpose(2, 0, 3, 1)
    out5 = _embed_gather(x5, phychem, n=n, s=s)
    return out5.transpose(2, 4, 0, 1, 3).reshape(n, s, D)


# speedup vs baseline: 5.2804x; 2.3744x over previous
import functools

import jax
import jax.numpy as jnp
from jax import lax
from jax.experimental import pallas as pl
from jax.experimental.pallas import tpu as pltpu
from jax.experimental.pallas import tpu_sc as plsc

D = 64
NC = 2
NS = 16
NW = NC * NS
L = 16
B = 128
NBH = 4


@functools.partial(jax.jit, static_argnames=("n", "s"))
def _embed_gather(x5, table, *, n, s):
    nbh_tot = n // B
    sp8 = (s + 7) // 8
    mesh = plsc.VectorSubcoreMesh(core_axis_name="c", subcore_axis_name="s")

    def body(x_hbm, table_hbm, out_hbm, idx_v, *rest):
        rows = rest[:NBH]
        tbuf = rest[NBH:2 * NBH]
        gsem = rest[2 * NBH:3 * NBH]
        osem = rest[3 * NBH:4 * NBH]
        wid = lax.axis_index("s") * NC + lax.axis_index("c")
        nh0 = wid * NBH

        pltpu.sync_copy(x_hbm.at[:, pl.ds(nh0, NBH)], idx_v)

        iota = lax.iota(jnp.int32, L)
        cperm = [jnp.bitwise_and(iota + k, L - 1) for k in range(L)]

        def g_start(s_, b):
            pltpu.async_copy(
                table_hbm.at[idx_v.at[s_ // 8, b, s_ % 8]], rows[b], gsem[b])

        def g_wait(s_, b):
            pltpu.make_async_copy(
                table_hbm.at[idx_v.at[s_ // 8, b, s_ % 8]], rows[b],
                gsem[b]).wait()

        def o_start(s_, b):
            pltpu.async_copy(
                tbuf[b], out_hbm.at[s_, :, nh0 + b], osem[b])

        def o_wait(s_, b):
            pltpu.make_async_copy(
                tbuf[b], out_hbm.at[s_, :, nh0 + b], osem[b]).wait()

        def transpose(b):
            @pl.loop(0, B // L)
            def _g(g):
                rv = iota + g * L
                for cb in range(D // L):
                    vals = [
                        plsc.load_gather(rows[b], [rv, cperm[k] + (cb * L)])
                        for k in range(L)
                    ]
                    for k in range(L):
                        cp = cperm[k]
                        plsc.store_scatter(
                            tbuf[b],
                            [jnp.right_shift(cp + (cb * L), 3),
                             jnp.bitwise_and(cp, 7), rv], vals[k])

        for b in range(NBH):
            g_start(0, b)

        @pl.loop(0, s)
        def _outer(s_):
            for b in range(NBH):
                g_wait(s_, b)

                @pl.when(s_ > 0)
                def _():
                    o_wait(s_ - 1, b)

                transpose(b)

                @pl.when(s_ < s - 1)
                def _():
                    g_start(s_ + 1, b)

                o_start(s_, b)

        for b in range(NBH):
            o_wait(s - 1, b)

    call = pl.kernel(
        body,
        out_type=jax.ShapeDtypeStruct((s, 8, nbh_tot, 8, B), jnp.float32),
        mesh=mesh,
        scratch_types=(
            [pltpu.VMEM((sp8, NBH, 8, B), jnp.int32)]
            + [pltpu.VMEM((B, D), jnp.float32) for _ in range(NBH)]
            + [pltpu.VMEM((8, 8, B), jnp.float32) for _ in range(NBH)]
            + [pltpu.SemaphoreType.DMA for _ in range(2 * NBH)]
        ),
        compiler_params=pltpu.CompilerParams(
            use_tc_tiling_on_sc=False, needs_layout_passes=False),
    )
    return call(x5, table)


def kernel(x, phychem):
    n, s = x.shape
    sp = (s + 7) // 8 * 8
    xp = jnp.pad(x, ((0, 0), (0, sp - s)))
    x5 = xp.reshape(n // B, B, sp // 8, 8).transpose(2, 0, 3, 1)
    out5 = _embed_gather(x5, phychem, n=n, s=s)
    return out5.transpose(2, 4, 0, 1, 3).reshape(n, s, D)


# speedup vs baseline: 5.2953x; 1.0028x over previous
import functools

import jax
import jax.numpy as jnp
from jax import lax
from jax.experimental import pallas as pl
from jax.experimental.pallas import tpu as pltpu
from jax.experimental.pallas import tpu_sc as plsc

D = 64
NC = 2
NS = 16
NW = NC * NS
L = 16
B = 128
NBH = 4


@functools.partial(jax.jit, static_argnames=("n", "s"))
def _embed_gather(x5, table, *, n, s):
    nbh_tot = n // B
    sp8 = (s + 7) // 8
    mesh = plsc.VectorSubcoreMesh(core_axis_name="c", subcore_axis_name="s")

    def body(x_hbm, table_hbm, out_hbm, idx_v, *rest):
        rows = rest[:NBH]
        tbuf = rest[NBH:2 * NBH]
        gsem = rest[2 * NBH:3 * NBH]
        osem = rest[3 * NBH:4 * NBH]
        wid = lax.axis_index("s") * NC + lax.axis_index("c")
        nh0 = wid * NBH

        pltpu.sync_copy(x_hbm.at[:, pl.ds(nh0, NBH)], idx_v)

        iota = lax.iota(jnp.int32, L)
        cperm = [jnp.bitwise_and(iota + k, L - 1) for k in range(L)]
        clop = [jnp.bitwise_and(cp, 7) for cp in cperm]

        def g_start(s_, b):
            pltpu.async_copy(
                table_hbm.at[idx_v.at[s_ // 8, b, s_ % 8]], rows[b], gsem[b])

        def g_wait(s_, b):
            pltpu.make_async_copy(
                table_hbm.at[idx_v.at[s_ // 8, b, s_ % 8]], rows[b],
                gsem[b]).wait()

        def o_start(s_, b):
            pltpu.async_copy(
                tbuf[b], out_hbm.at[s_, :, nh0 + b], osem[b])

        def o_wait(s_, b):
            pltpu.make_async_copy(
                tbuf[b], out_hbm.at[s_, :, nh0 + b], osem[b]).wait()

        def transpose(b):
            @pl.loop(0, B // L)
            def _g(g):
                rv = iota + g * L
                for cb in range(D // L):
                    vals = [
                        plsc.load_gather(rows[b], [rv, cperm[k] + (cb * L)])
                        for k in range(L)
                    ]
                    for k in range(L):
                        plsc.store_scatter(
                            tbuf[b],
                            [jnp.right_shift(cperm[k], 3) + (2 * cb),
                             clop[k], rv], vals[k])

        for b in range(NBH):
            g_start(0, b)

        @pl.loop(0, s)
        def _outer(s_):
            for b in range(NBH):
                g_wait(s_, b)

                @pl.when(s_ > 0)
                def _():
                    o_wait(s_ - 1, b)

                transpose(b)

                @pl.when(s_ < s - 1)
                def _():
                    g_start(s_ + 1, b)

                o_start(s_, b)

        for b in range(NBH):
            o_wait(s - 1, b)

    call = pl.kernel(
        body,
        out_type=jax.ShapeDtypeStruct((s, 8, nbh_tot, 8, B), jnp.float32),
        mesh=mesh,
        scratch_types=(
            [pltpu.VMEM((sp8, NBH, 8, B), jnp.int32)]
            + [pltpu.VMEM((B, D), jnp.float32) for _ in range(NBH)]
            + [pltpu.VMEM((8, 8, B), jnp.float32) for _ in range(NBH)]
            + [pltpu.SemaphoreType.DMA for _ in range(2 * NBH)]
        ),
        compiler_params=pltpu.CompilerParams(
            use_tc_tiling_on_sc=False, needs_layout_passes=False),
    )
    return call(x5, table)


def kernel(x, phychem):
    n, s = x.shape
    sp = (s + 7) // 8 * 8
    xp = jnp.pad(x, ((0, 0), (0, sp - s)))
    x5 = xp.reshape(n // B, B, sp // 8, 8).transpose(2, 0, 3, 1)
    out5 = _embed_gather(x5, phychem, n=n, s=s)
    return out5.transpose(2, 4, 0, 1, 3).reshape(n, s, D)
